# baseline (speedup 1.0000x reference)
import functools

import jax
import jax.numpy as jnp
from jax import lax
from jax.experimental import pallas as pl
from jax.experimental.pallas import tpu as pltpu
from jax.experimental.pallas import tpu_sc as plsc

N_NODES = 10000
N_EDGES = 320000
FEAT = 128
DIM_RBF = 16

NC = 2
NS = 16
NW = NC * NS
LANES = 128
ROWS = N_EDGES // LANES
N_PAD = 10112
ROWS_PER_TILE = N_PAD // NS

K_SPLIT = 2
HROWS = ROWS // K_SPLIT
WROWS = HROWS // NW
REM = HROWS % NW
PAIRS = WROWS // 2
TAIL = WROWS % 2

_BE = 6400
_HBLOCKS = (N_EDGES // K_SPLIT) // _BE


def _msg_body(m_ref, rbft_ref, w_ref, out_ref):
    e = lax.dot_general(rbft_ref[...], w_ref[...],
                        dimension_numbers=(((0,), (1,)), ((), ())),
                        preferred_element_type=jnp.float32)
    out_ref[...] = e * m_ref[...]


def _scatter_body(msg_hbm, dst_hbm, out_hbm,
                  msg_0, msg_1, msg_2, idx_0, idx_1, idx_2, idx_l, acc_sh,
                  lsem_0, lsem_1, lsem_2, ssem_0, ssem_1, ssem_2):
    cid = lax.axis_index("c")
    sid = lax.axis_index("s")
    w = sid * NC + cid
    base = w * WROWS + jnp.minimum(w, REM)

    bufs = (msg_0, msg_1, msg_2)
    idxs = (idx_0, idx_1, idx_2)
    lsems = (lsem_0, lsem_1, lsem_2)
    ssems = (ssem_0, ssem_1, ssem_2)

    def zrow(i, _):
        for j in range(FEAT // 16):
            msg_0[i, pl.ds(j * 16, 16)] = jnp.zeros((16,), jnp.float32)
        return _

    lax.fori_loop(0, LANES, zrow, None)
    arow = sid * ROWS_PER_TILE
    for r in range(ROWS_PER_TILE // LANES):
        pltpu.sync_copy(msg_0, acc_sh.at[pl.ds(arow + r * LANES, LANES)])
    tail_rows = ROWS_PER_TILE % LANES
    if tail_rows:
        pltpu.sync_copy(
            msg_0.at[pl.ds(0, tail_rows)],
            acc_sh.at[pl.ds(arow + ROWS_PER_TILE - tail_rows, tail_rows)],
        )
    plsc.subcore_barrier()

    def chunk(s):
        return msg_hbm.at[pl.ds((base + s) * LANES, LANES)]

    def load(s, b):
        pltpu.async_copy(chunk(s), bufs[b], lsems[b])
        pltpu.async_copy(dst_hbm.at[base + s], idxs[b], lsems[b])

    def wait_load(s, b):
        pltpu.make_async_copy(chunk(s), bufs[b], lsems[b]).wait()
        pltpu.make_async_copy(dst_hbm.at[base + s], idxs[b], lsems[b]).wait()

    def scatter(b):
        return pltpu.async_copy(bufs[b], acc_sh.at[idxs[b].at[0]], ssems[b],
                                add=True)

    def wait_scatter(b):
        pltpu.make_async_copy(bufs[b], acc_sh.at[idxs[b].at[0]],
                              ssems[b]).wait()

    load(0, 0)
    load(1, 1)

    def step3(g, _):
        for b in range(3):
            s = 3 * g + b
            wait_load(s, b)
            scatter(b)
            bp = (b + 2) % 3

            @pl.when(s >= 1)
            def _():
                wait_scatter(bp)

            @pl.when(s + 2 < WROWS)
            def _():
                load(s + 2, bp)
        return _

    lax.fori_loop(0, WROWS // 3, step3, None)
    wait_scatter((WROWS - 1) % 3)

    @pl.when(w < REM)
    def _():
        row = base + WROWS
        pltpu.sync_copy(msg_hbm.at[pl.ds(row * LANES, LANES)], msg_0)
        pltpu.sync_copy(dst_hbm.at[row], idx_l)
        pltpu.sync_copy(msg_0, acc_sh.at[idx_l.at[0]], add=True)

    plsc.subcore_barrier()

    pltpu.sync_copy(
        acc_sh.at[pl.ds(sid * ROWS_PER_TILE, ROWS_PER_TILE)],
        out_hbm.at[cid, pl.ds(sid * ROWS_PER_TILE, ROWS_PER_TILE)],
    )


def _dot_nt(a, w_ref):
    return lax.dot_general(a, w_ref[...],
                           dimension_numbers=(((1,), (1,)), ((), ())),
                           preferred_element_type=jnp.float32)


def _mlp_body(*refs):
    acc_refs = refs[:K_SPLIT]
    w1_ref, b1_ref, w2_ref, b2_ref, w3_ref, b3_ref, wo_ref, out_ref = refs[K_SPLIT:]
    a = acc_refs[0][0] + acc_refs[0][1]
    for acc in acc_refs[1:]:
        a = a + acc[0] + acc[1]
    a = _dot_nt(a, w1_ref) + b1_ref[...]
    a = a * jax.nn.sigmoid(a)
    a = _dot_nt(a, w2_ref) + b2_ref[...]
    a = a * jax.nn.sigmoid(a)
    a = _dot_nt(a, w3_ref) + b3_ref[...]
    a = a * jax.nn.sigmoid(a)
    out_ref[...] = _dot_nt(a, wo_ref)[:N_NODES]


def kernel(m_ji, rbf_ji, atom_edge_index, W_rbf, W1, b1, W2, b2, W3, b3, W_out):
    rbf_t = rbf_ji.T
    dst3d = atom_edge_index[1].astype(jnp.int32).reshape(ROWS, 1, LANES)
    mesh = plsc.VectorSubcoreMesh(core_axis_name="c", subcore_axis_name="s")

    def msg_half(h):
        off = h * _HBLOCKS
        return pl.pallas_call(
            _msg_body,
            grid=(_HBLOCKS,),
            in_specs=[
                pl.BlockSpec((_BE, FEAT), lambda i: (i + off, 0)),
                pl.BlockSpec((DIM_RBF, _BE), lambda i: (0, i + off)),
                pl.BlockSpec((FEAT, DIM_RBF), lambda i: (0, 0)),
            ],
            out_specs=pl.BlockSpec((_BE, FEAT), lambda i: (i, 0)),
            out_shape=jax.ShapeDtypeStruct((N_EDGES // K_SPLIT, FEAT), jnp.float32),
        )(m_ji, rbf_t, W_rbf)

    scatter = functools.partial(
        pl.kernel,
        out_type=jax.ShapeDtypeStruct((NC, N_PAD, FEAT), jnp.float32),
        mesh=mesh,
        scratch_types=[
            pltpu.VMEM((LANES, FEAT), jnp.float32),
            pltpu.VMEM((LANES, FEAT), jnp.float32),
            pltpu.VMEM((LANES, FEAT), jnp.float32),
            pltpu.VMEM((1, LANES), jnp.int32),
            pltpu.VMEM((1, LANES), jnp.int32),
            pltpu.VMEM((1, LANES), jnp.int32),
            pltpu.VMEM((1, LANES), jnp.int32),
            pltpu.VMEM_SHARED((N_PAD, FEAT), jnp.float32),
            pltpu.SemaphoreType.DMA,
            pltpu.SemaphoreType.DMA,
            pltpu.SemaphoreType.DMA,
            pltpu.SemaphoreType.DMA,
            pltpu.SemaphoreType.DMA,
            pltpu.SemaphoreType.DMA,
        ],
    )(_scatter_body)

    partials = []
    for h in range(K_SPLIT):
        msg_h = msg_half(h)
        dst_h = lax.slice_in_dim(dst3d, h * HROWS, (h + 1) * HROWS, axis=0)
        partials.append(scatter(msg_h, dst_h))

    out = pl.pallas_call(
        _mlp_body,
        in_specs=[pl.BlockSpec(memory_space=pltpu.MemorySpace.VMEM)] * (K_SPLIT + 7),
        out_specs=pl.BlockSpec(memory_space=pltpu.MemorySpace.VMEM),
        out_shape=jax.ShapeDtypeStruct((N_NODES, 1), jnp.float32),
    )(*partials, W1, b1.reshape(1, FEAT), W2,
      b2.reshape(1, FEAT), W3, b3.reshape(1, FEAT), W_out)
    return out


# speedup vs baseline: 1.0231x; 1.0231x over previous
import functools

import jax
import jax.numpy as jnp
from jax import lax
from jax.experimental import pallas as pl
from jax.experimental.pallas import tpu as pltpu
from jax.experimental.pallas import tpu_sc as plsc

N_NODES = 10000
N_EDGES = 320000
FEAT = 128
DIM_RBF = 16

NC = 2
NS = 16
NW = NC * NS
LANES = 128
ROWS = N_EDGES // LANES
N_PAD = 10112
ROWS_PER_TILE = N_PAD // NS

K_SPLIT = 2
HROWS = ROWS // K_SPLIT
WROWS = HROWS // NW
REM = HROWS % NW
PAIRS = WROWS // 2
TAIL = WROWS % 2

_BE = 16000
_HBLOCKS = (N_EDGES // K_SPLIT) // _BE


def _msg_body(m_ref, rbft_ref, wt_ref, out_ref):
    e = lax.dot_general(rbft_ref[...], wt_ref[...],
                        dimension_numbers=(((0,), (0,)), ((), ())),
                        preferred_element_type=jnp.float32)
    out_ref[...] = e * m_ref[...]


def _scatter_body(msg_hbm, dst_hbm, out_hbm,
                  msg_0, msg_1, msg_2, idx_0, idx_1, idx_2, idx_l, acc_sh,
                  lsem_0, lsem_1, lsem_2, ssem_0, ssem_1, ssem_2):
    cid = lax.axis_index("c")
    sid = lax.axis_index("s")
    w = sid * NC + cid
    base = w * WROWS + jnp.minimum(w, REM)

    bufs = (msg_0, msg_1, msg_2)
    idxs = (idx_0, idx_1, idx_2)
    lsems = (lsem_0, lsem_1, lsem_2)
    ssems = (ssem_0, ssem_1, ssem_2)

    def zrow(i, _):
        for j in range(FEAT // 16):
            msg_0[i, pl.ds(j * 16, 16)] = jnp.zeros((16,), jnp.float32)
        return _

    lax.fori_loop(0, LANES, zrow, None)
    arow = sid * ROWS_PER_TILE
    for r in range(ROWS_PER_TILE // LANES):
        pltpu.sync_copy(msg_0, acc_sh.at[pl.ds(arow + r * LANES, LANES)])
    tail_rows = ROWS_PER_TILE % LANES
    if tail_rows:
        pltpu.sync_copy(
            msg_0.at[pl.ds(0, tail_rows)],
            acc_sh.at[pl.ds(arow + ROWS_PER_TILE - tail_rows, tail_rows)],
        )
    plsc.subcore_barrier()

    def chunk(s):
        return msg_hbm.at[pl.ds((base + s) * LANES, LANES)]

    def load(s, b):
        pltpu.async_copy(chunk(s), bufs[b], lsems[b])
        pltpu.async_copy(dst_hbm.at[base + s], idxs[b], lsems[b])

    def wait_load(s, b):
        pltpu.make_async_copy(chunk(s), bufs[b], lsems[b]).wait()
        pltpu.make_async_copy(dst_hbm.at[base + s], idxs[b], lsems[b]).wait()

    def scatter(b):
        return pltpu.async_copy(bufs[b], acc_sh.at[idxs[b].at[0]], ssems[b],
                                add=True)

    def wait_scatter(b):
        pltpu.make_async_copy(bufs[b], acc_sh.at[idxs[b].at[0]],
                              ssems[b]).wait()

    load(0, 0)
    load(1, 1)

    def step3(g, _):
        for b in range(3):
            s = 3 * g + b
            wait_load(s, b)
            scatter(b)
            bp = (b + 2) % 3

            @pl.when(s >= 1)
            def _():
                wait_scatter(bp)

            @pl.when(s + 2 < WROWS)
            def _():
                load(s + 2, bp)
        return _

    lax.fori_loop(0, WROWS // 3, step3, None)
    wait_scatter((WROWS - 1) % 3)

    @pl.when(w < REM)
    def _():
        row = base + WROWS
        pltpu.sync_copy(msg_hbm.at[pl.ds(row * LANES, LANES)], msg_0)
        pltpu.sync_copy(dst_hbm.at[row], idx_l)
        pltpu.sync_copy(msg_0, acc_sh.at[idx_l.at[0]], add=True)

    plsc.subcore_barrier()

    pltpu.sync_copy(
        acc_sh.at[pl.ds(sid * ROWS_PER_TILE, ROWS_PER_TILE)],
        out_hbm.at[cid, pl.ds(sid * ROWS_PER_TILE, ROWS_PER_TILE)],
    )


def _mlp_body(*refs):
    acc_refs = refs[:K_SPLIT]
    w1_ref, b1_ref, w2_ref, b2_ref, w3_ref, b3_ref, wo_ref, out_ref = refs[K_SPLIT:]
    a = acc_refs[0][0] + acc_refs[0][1]
    for acc in acc_refs[1:]:
        a = a + acc[0] + acc[1]
    a = jnp.dot(a, w1_ref[...], preferred_element_type=jnp.float32) + b1_ref[...]
    a = a * jax.nn.sigmoid(a)
    a = jnp.dot(a, w2_ref[...], preferred_element_type=jnp.float32) + b2_ref[...]
    a = a * jax.nn.sigmoid(a)
    a = jnp.dot(a, w3_ref[...], preferred_element_type=jnp.float32) + b3_ref[...]
    a = a * jax.nn.sigmoid(a)
    out_ref[...] = jnp.dot(a, wo_ref[...], preferred_element_type=jnp.float32)[:N_NODES]


def kernel(m_ji, rbf_ji, atom_edge_index, W_rbf, W1, b1, W2, b2, W3, b3, W_out):
    rbf_t = rbf_ji.T
    w_rbf_t = W_rbf.T
    dst3d = atom_edge_index[1].astype(jnp.int32).reshape(ROWS, 1, LANES)
    mesh = plsc.VectorSubcoreMesh(core_axis_name="c", subcore_axis_name="s")

    def msg_half(h):
        off = h * _HBLOCKS
        return pl.pallas_call(
            _msg_body,
            grid=(_HBLOCKS,),
            in_specs=[
                pl.BlockSpec((_BE, FEAT), lambda i: (i + off, 0)),
                pl.BlockSpec((DIM_RBF, _BE), lambda i: (0, i + off)),
                pl.BlockSpec((DIM_RBF, FEAT), lambda i: (0, 0)),
            ],
            out_specs=pl.BlockSpec((_BE, FEAT), lambda i: (i, 0)),
            out_shape=jax.ShapeDtypeStruct((N_EDGES // K_SPLIT, FEAT), jnp.float32),
        )(m_ji, rbf_t, w_rbf_t)

    scatter = functools.partial(
        pl.kernel,
        out_type=jax.ShapeDtypeStruct((NC, N_PAD, FEAT), jnp.float32),
        mesh=mesh,
        scratch_types=[
            pltpu.VMEM((LANES, FEAT), jnp.float32),
            pltpu.VMEM((LANES, FEAT), jnp.float32),
            pltpu.VMEM((LANES, FEAT), jnp.float32),
            pltpu.VMEM((1, LANES), jnp.int32),
            pltpu.VMEM((1, LANES), jnp.int32),
            pltpu.VMEM((1, LANES), jnp.int32),
            pltpu.VMEM((1, LANES), jnp.int32),
            pltpu.VMEM_SHARED((N_PAD, FEAT), jnp.float32),
            pltpu.SemaphoreType.DMA,
            pltpu.SemaphoreType.DMA,
            pltpu.SemaphoreType.DMA,
            pltpu.SemaphoreType.DMA,
            pltpu.SemaphoreType.DMA,
            pltpu.SemaphoreType.DMA,
        ],
    )(_scatter_body)

    partials = []
    for h in range(K_SPLIT):
        msg_h = msg_half(h)
        dst_h = lax.slice_in_dim(dst3d, h * HROWS, (h + 1) * HROWS, axis=0)
        partials.append(scatter(msg_h, dst_h))

    out = pl.pallas_call(
        _mlp_body,
        in_specs=[pl.BlockSpec(memory_space=pltpu.MemorySpace.VMEM)] * (K_SPLIT + 7),
        out_specs=pl.BlockSpec(memory_space=pltpu.MemorySpace.VMEM),
        out_shape=jax.ShapeDtypeStruct((N_NODES, 1), jnp.float32),
    )(*partials, W1.T, b1.reshape(1, FEAT), W2.T,
      b2.reshape(1, FEAT), W3.T, b3.reshape(1, FEAT), W_out.T)
    return out


# speedup vs baseline: 1.0249x; 1.0017x over previous
import functools

import jax
import jax.numpy as jnp
from jax import lax
from jax.experimental import pallas as pl
from jax.experimental.pallas import tpu as pltpu
from jax.experimental.pallas import tpu_sc as plsc

N_NODES = 10000
N_EDGES = 320000
FEAT = 128
DIM_RBF = 16

NC = 2
NS = 16
NW = NC * NS
LANES = 128
ROWS = N_EDGES // LANES
N_PAD = 10112
ROWS_PER_TILE = N_PAD // NS

K_SPLIT = 2
HROWS = ROWS // K_SPLIT
WROWS = HROWS // NW
REM = HROWS % NW

_BE = 16000
_HBLOCKS = (N_EDGES // K_SPLIT) // _BE


def _msg_body(m_ref, rbft_ref, wt_ref, out_ref):
    e = lax.dot_general(rbft_ref[...], wt_ref[...],
                        dimension_numbers=(((0,), (0,)), ((), ())),
                        preferred_element_type=jnp.float32)
    out_ref[...] = e * m_ref[...]


def _scatter_body(msg_hbm, dst_hbm, out_hbm,
                  msg_0, msg_1, msg_2, idx_0, idx_1, idx_2, idx_l, acc_sh,
                  lsem_0, lsem_1, lsem_2, ssem_0, ssem_1, ssem_2):
    cid = lax.axis_index("c")
    sid = lax.axis_index("s")
    w = sid * NC + cid
    base = w * WROWS + jnp.minimum(w, REM)

    bufs = (msg_0, msg_1, msg_2)
    idxs = (idx_0, idx_1, idx_2)
    lsems = (lsem_0, lsem_1, lsem_2)
    ssems = (ssem_0, ssem_1, ssem_2)

    def zrow(i, _):
        for j in range(FEAT // 16):
            msg_0[i, pl.ds(j * 16, 16)] = jnp.zeros((16,), jnp.float32)
        return _

    lax.fori_loop(0, LANES, zrow, None)
    arow = sid * ROWS_PER_TILE
    for r in range(ROWS_PER_TILE // LANES):
        pltpu.sync_copy(msg_0, acc_sh.at[pl.ds(arow + r * LANES, LANES)])
    tail_rows = ROWS_PER_TILE % LANES
    if tail_rows:
        pltpu.sync_copy(
            msg_0.at[pl.ds(0, tail_rows)],
            acc_sh.at[pl.ds(arow + ROWS_PER_TILE - tail_rows, tail_rows)],
        )
    plsc.subcore_barrier()

    def chunk(s):
        return msg_hbm.at[pl.ds((base + s) * LANES, LANES)]

    def load(s, b):
        pltpu.async_copy(chunk(s), bufs[b], lsems[b])
        pltpu.async_copy(dst_hbm.at[base + s], idxs[b], lsems[b])

    def wait_load(s, b):
        pltpu.make_async_copy(chunk(s), bufs[b], lsems[b]).wait()
        pltpu.make_async_copy(dst_hbm.at[base + s], idxs[b], lsems[b]).wait()

    def scatter(b):
        return pltpu.async_copy(bufs[b], acc_sh.at[idxs[b].at[0]], ssems[b],
                                add=True)

    def wait_scatter(b):
        pltpu.make_async_copy(bufs[b], acc_sh.at[idxs[b].at[0]],
                              ssems[b]).wait()

    load(0, 0)
    load(1, 1)

    def step3(g, _):
        for b in range(3):
            s = 3 * g + b
            wait_load(s, b)
            scatter(b)
            bp = (b + 2) % 3

            @pl.when(s >= 1)
            def _():
                wait_scatter(bp)

            @pl.when(s + 2 < WROWS)
            def _():
                load(s + 2, bp)
        return _

    lax.fori_loop(0, WROWS // 3, step3, None)
    wait_scatter((WROWS - 1) % 3)

    @pl.when(w < REM)
    def _():
        row = base + WROWS
        pltpu.sync_copy(msg_hbm.at[pl.ds(row * LANES, LANES)], msg_0)
        pltpu.sync_copy(dst_hbm.at[row], idx_l)
        pltpu.sync_copy(msg_0, acc_sh.at[idx_l.at[0]], add=True)

    plsc.subcore_barrier()

    pltpu.sync_copy(
        acc_sh.at[pl.ds(sid * ROWS_PER_TILE, ROWS_PER_TILE)],
        out_hbm.at[cid, pl.ds(sid * ROWS_PER_TILE, ROWS_PER_TILE)],
    )


def _mlp_body(*refs):
    acc_refs = refs[:K_SPLIT]
    w1_ref, b1_ref, w2_ref, b2_ref, w3_ref, b3_ref, wo_ref, out_ref = refs[K_SPLIT:]
    a = acc_refs[0][0] + acc_refs[0][1]
    for acc in acc_refs[1:]:
        a = a + acc[0] + acc[1]
    a = jnp.dot(a, w1_ref[...], preferred_element_type=jnp.float32) + b1_ref[...]
    a = a * jax.nn.sigmoid(a)
    a = jnp.dot(a, w2_ref[...], preferred_element_type=jnp.float32) + b2_ref[...]
    a = a * jax.nn.sigmoid(a)
    a = jnp.dot(a, w3_ref[...], preferred_element_type=jnp.float32) + b3_ref[...]
    a = a * jax.nn.sigmoid(a)
    out_ref[...] = jnp.dot(a, wo_ref[...], preferred_element_type=jnp.float32)[:N_NODES]


def kernel(m_ji, rbf_ji, atom_edge_index, W_rbf, W1, b1, W2, b2, W3, b3, W_out):
    rbf_t = rbf_ji.T
    w_rbf_t = W_rbf.T
    dst3d = atom_edge_index[1].astype(jnp.int32).reshape(ROWS, 1, LANES)
    mesh = plsc.VectorSubcoreMesh(core_axis_name="c", subcore_axis_name="s")

    def msg_half(h):
        off = h * _HBLOCKS
        return pl.pallas_call(
            _msg_body,
            grid=(_HBLOCKS,),
            in_specs=[
                pl.BlockSpec((_BE, FEAT), lambda i: (i + off, 0)),
                pl.BlockSpec((DIM_RBF, _BE), lambda i: (0, i + off)),
                pl.BlockSpec((DIM_RBF, FEAT), lambda i: (0, 0)),
            ],
            out_specs=pl.BlockSpec((_BE, FEAT), lambda i: (i, 0)),
            out_shape=jax.ShapeDtypeStruct((N_EDGES // K_SPLIT, FEAT), jnp.float32),
        )(m_ji, rbf_t, w_rbf_t)

    scatter = functools.partial(
        pl.kernel,
        out_type=jax.ShapeDtypeStruct((NC, N_PAD, FEAT), jnp.float32),
        mesh=mesh,
        scratch_types=[
            pltpu.VMEM((LANES, FEAT), jnp.float32),
            pltpu.VMEM((LANES, FEAT), jnp.float32),
            pltpu.VMEM((LANES, FEAT), jnp.float32),
            pltpu.VMEM((1, LANES), jnp.int32),
            pltpu.VMEM((1, LANES), jnp.int32),
            pltpu.VMEM((1, LANES), jnp.int32),
            pltpu.VMEM((1, LANES), jnp.int32),
            pltpu.VMEM_SHARED((N_PAD, FEAT), jnp.float32),
            pltpu.SemaphoreType.DMA,
            pltpu.SemaphoreType.DMA,
            pltpu.SemaphoreType.DMA,
            pltpu.SemaphoreType.DMA,
            pltpu.SemaphoreType.DMA,
            pltpu.SemaphoreType.DMA,
        ],
    )(_scatter_body)

    partials = []
    for h in range(K_SPLIT):
        msg_h = msg_half(h)
        dst_h = lax.slice_in_dim(dst3d, h * HROWS, (h + 1) * HROWS, axis=0)
        partials.append(scatter(msg_h, dst_h))

    out = pl.pallas_call(
        _mlp_body,
        in_specs=[pl.BlockSpec(memory_space=pltpu.MemorySpace.VMEM)] * (K_SPLIT + 7),
        out_specs=pl.BlockSpec(memory_space=pltpu.MemorySpace.VMEM),
        out_shape=jax.ShapeDtypeStruct((N_NODES, 1), jnp.float32),
    )(*partials, W1.T, b1.reshape(1, FEAT), W2.T,
      b2.reshape(1, FEAT), W3.T, b3.reshape(1, FEAT), W_out.T)
    return out
